# baseline (device time: 14175 ns/iter reference)
import jax
import jax.numpy as jnp
from jax import lax
from jax.experimental import pallas as pl
from jax.experimental.pallas import tpu as pltpu

N_DEV = 4
N_CHUNK = 2


def kernel(table, idx):
    v_per, d = table.shape
    n = idx.shape[0]
    h = n // N_CHUNK
    idx2 = idx.reshape(n, 1)

    def body(table_ref, idx_ref, out_ref, comm_ref, send_sems, recv_sems):
        my_pos = lax.axis_index("i")
        partner1 = lax.bitwise_xor(my_pos, 1)
        partner2 = 3 - my_pos

        barrier_sem = pltpu.get_barrier_semaphore()
        for nbr in (partner1, partner2):
            pl.semaphore_signal(
                barrier_sem, inc=1,
                device_id=(nbr,), device_id_type=pl.DeviceIdType.MESH,
            )
        pl.semaphore_wait(barrier_sem, 2)

        OWN, R1, SUM, R2 = 0, 1, 2, 3

        tbl = table_ref[...].astype(jnp.bfloat16)
        ex1 = []
        for c in range(N_CHUNK):
            rows = pl.ds(c * h, h)
            local = idx_ref[rows, :] - my_pos * v_per
            cols = lax.broadcasted_iota(jnp.int32, (h, v_per), 1)
            onehot = (cols == local).astype(jnp.bfloat16)
            pc = jnp.dot(
                onehot, tbl, preferred_element_type=jnp.float32
            ).astype(jnp.bfloat16)
            comm_ref[OWN, rows, :] = pc
            rdma = pltpu.make_async_remote_copy(
                src_ref=comm_ref.at[OWN, rows, :],
                dst_ref=comm_ref.at[R1, rows, :],
                send_sem=send_sems.at[0, c],
                recv_sem=recv_sems.at[0, c],
                device_id=(partner1,),
                device_id_type=pl.DeviceIdType.MESH,
            )
            rdma.start()
            ex1.append(rdma)

        ex2 = []
        for c in range(N_CHUNK):
            rows = pl.ds(c * h, h)
            ex1[c].wait_recv()
            comm_ref[SUM, rows, :] = (
                comm_ref[OWN, rows, :] + comm_ref[R1, rows, :]
            )
            rdma = pltpu.make_async_remote_copy(
                src_ref=comm_ref.at[SUM, rows, :],
                dst_ref=comm_ref.at[R2, rows, :],
                send_sem=send_sems.at[1, c],
                recv_sem=recv_sems.at[1, c],
                device_id=(partner2,),
                device_id_type=pl.DeviceIdType.MESH,
            )
            rdma.start()
            ex2.append(rdma)

        for c in range(N_CHUNK):
            rows = pl.ds(c * h, h)
            ex2[c].wait_recv()
            out_ref[rows, :] = comm_ref[SUM, rows, :] + comm_ref[R2, rows, :]

        for rdma in ex1 + ex2:
            rdma.wait_send()

    return pl.pallas_call(
        body,
        out_shape=jax.ShapeDtypeStruct((n, d), jnp.bfloat16),
        in_specs=[
            pl.BlockSpec(memory_space=pltpu.VMEM),
            pl.BlockSpec(memory_space=pltpu.VMEM),
        ],
        out_specs=pl.BlockSpec(memory_space=pltpu.VMEM),
        scratch_shapes=[
            pltpu.VMEM((4, n, d), jnp.bfloat16),
            pltpu.SemaphoreType.DMA((2, N_CHUNK)),
            pltpu.SemaphoreType.DMA((2, N_CHUNK)),
        ],
        compiler_params=pltpu.CompilerParams(collective_id=0),
    )(table, idx2)


# device time: 14067 ns/iter; 1.0077x vs baseline; 1.0077x over previous
import jax
import jax.numpy as jnp
from jax import lax
from jax.experimental import pallas as pl
from jax.experimental.pallas import tpu as pltpu

N_DEV = 4
N_CHUNK = 2


def kernel(table, idx):
    v_per, d = table.shape
    n = idx.shape[0]
    h = n // N_CHUNK

    def body(table_ref, idx_ref, out_ref, comm_ref, send_sems, recv_sems):
        my_pos = lax.axis_index("i")
        partner1 = lax.bitwise_xor(my_pos, 1)
        partner2 = 3 - my_pos

        barrier_sem = pltpu.get_barrier_semaphore()
        for nbr in (partner1, partner2):
            pl.semaphore_signal(
                barrier_sem, inc=1,
                device_id=(nbr,), device_id_type=pl.DeviceIdType.MESH,
            )
        pl.semaphore_wait(barrier_sem, 2)

        OWN, R1, SUM, R2 = 0, 1, 2, 3

        tbl = table_ref[...].astype(jnp.bfloat16)
        idx_row = jnp.reshape(idx_ref[...], (1, n)) - my_pos * v_per
        ex1 = []
        for c in range(N_CHUNK):
            rows = pl.ds(c * h, h)
            vrows = lax.broadcasted_iota(jnp.int32, (v_per, h), 0)
            onehot_t = (vrows == idx_row[:, c * h:(c + 1) * h]).astype(
                jnp.bfloat16
            )
            pc = lax.dot_general(
                onehot_t, tbl, (((0,), (0,)), ((), ())),
                preferred_element_type=jnp.float32,
            ).astype(jnp.bfloat16)
            comm_ref[OWN, rows, :] = pc
            rdma = pltpu.make_async_remote_copy(
                src_ref=comm_ref.at[OWN, rows, :],
                dst_ref=comm_ref.at[R1, rows, :],
                send_sem=send_sems.at[0, c],
                recv_sem=recv_sems.at[0, c],
                device_id=(partner1,),
                device_id_type=pl.DeviceIdType.MESH,
            )
            rdma.start()
            ex1.append(rdma)

        ex2 = []
        for c in range(N_CHUNK):
            rows = pl.ds(c * h, h)
            ex1[c].wait_recv()
            comm_ref[SUM, rows, :] = (
                comm_ref[OWN, rows, :] + comm_ref[R1, rows, :]
            )
            rdma = pltpu.make_async_remote_copy(
                src_ref=comm_ref.at[SUM, rows, :],
                dst_ref=comm_ref.at[R2, rows, :],
                send_sem=send_sems.at[1, c],
                recv_sem=recv_sems.at[1, c],
                device_id=(partner2,),
                device_id_type=pl.DeviceIdType.MESH,
            )
            rdma.start()
            ex2.append(rdma)

        for c in range(N_CHUNK):
            rows = pl.ds(c * h, h)
            ex2[c].wait_recv()
            out_ref[rows, :] = comm_ref[SUM, rows, :] + comm_ref[R2, rows, :]

        for rdma in ex1 + ex2:
            rdma.wait_send()

    return pl.pallas_call(
        body,
        out_shape=jax.ShapeDtypeStruct((n, d), jnp.bfloat16),
        in_specs=[
            pl.BlockSpec(memory_space=pltpu.VMEM),
            pl.BlockSpec(memory_space=pltpu.VMEM),
        ],
        out_specs=pl.BlockSpec(memory_space=pltpu.VMEM),
        scratch_shapes=[
            pltpu.VMEM((4, n, d), jnp.bfloat16),
            pltpu.SemaphoreType.DMA((2, N_CHUNK)),
            pltpu.SemaphoreType.DMA((2, N_CHUNK)),
        ],
        compiler_params=pltpu.CompilerParams(collective_id=0),
    )(table, idx)


# device time: 13180 ns/iter; 1.0755x vs baseline; 1.0673x over previous
import jax
import jax.numpy as jnp
from jax import lax
from jax.experimental import pallas as pl
from jax.experimental.pallas import tpu as pltpu

N_DEV = 4
N_CHUNK = 4


def kernel(table, idx):
    v_per, d = table.shape
    n = idx.shape[0]
    h = n // N_CHUNK

    def body(table_ref, idx_ref, out_ref, comm_ref, send_sems, recv_sems):
        my_pos = lax.axis_index("i")
        partner1 = lax.bitwise_xor(my_pos, 1)
        partner2 = 3 - my_pos

        barrier_sem = pltpu.get_barrier_semaphore()
        for nbr in (partner1, partner2):
            pl.semaphore_signal(
                barrier_sem, inc=1,
                device_id=(nbr,), device_id_type=pl.DeviceIdType.MESH,
            )
        pl.semaphore_wait(barrier_sem, 2)

        OWN, R1, SUM, R2 = 0, 1, 2, 3

        tbl = table_ref[...].astype(jnp.bfloat16)
        idx_row = jnp.reshape(idx_ref[...], (1, n)) - my_pos * v_per
        ex1 = []
        for c in range(N_CHUNK):
            rows = pl.ds(c * h, h)
            vrows = lax.broadcasted_iota(jnp.int32, (v_per, h), 0)
            onehot_t = (vrows == idx_row[:, c * h:(c + 1) * h]).astype(
                jnp.bfloat16
            )
            pc = lax.dot_general(
                onehot_t, tbl, (((0,), (0,)), ((), ())),
                preferred_element_type=jnp.float32,
            ).astype(jnp.bfloat16)
            comm_ref[OWN, rows, :] = pc
            rdma = pltpu.make_async_remote_copy(
                src_ref=comm_ref.at[OWN, rows, :],
                dst_ref=comm_ref.at[R1, rows, :],
                send_sem=send_sems.at[0, c],
                recv_sem=recv_sems.at[0, c],
                device_id=(partner1,),
                device_id_type=pl.DeviceIdType.MESH,
            )
            rdma.start()
            ex1.append(rdma)

        ex2 = []
        for c in range(N_CHUNK):
            rows = pl.ds(c * h, h)
            ex1[c].wait_recv()
            comm_ref[SUM, rows, :] = (
                comm_ref[OWN, rows, :] + comm_ref[R1, rows, :]
            )
            rdma = pltpu.make_async_remote_copy(
                src_ref=comm_ref.at[SUM, rows, :],
                dst_ref=comm_ref.at[R2, rows, :],
                send_sem=send_sems.at[1, c],
                recv_sem=recv_sems.at[1, c],
                device_id=(partner2,),
                device_id_type=pl.DeviceIdType.MESH,
            )
            rdma.start()
            ex2.append(rdma)

        for c in range(N_CHUNK):
            rows = pl.ds(c * h, h)
            ex2[c].wait_recv()
            out_ref[rows, :] = comm_ref[SUM, rows, :] + comm_ref[R2, rows, :]

        for rdma in ex1 + ex2:
            rdma.wait_send()

    return pl.pallas_call(
        body,
        out_shape=jax.ShapeDtypeStruct((n, d), jnp.bfloat16),
        in_specs=[
            pl.BlockSpec(memory_space=pltpu.VMEM),
            pl.BlockSpec(memory_space=pltpu.VMEM),
        ],
        out_specs=pl.BlockSpec(memory_space=pltpu.VMEM),
        scratch_shapes=[
            pltpu.VMEM((4, n, d), jnp.bfloat16),
            pltpu.SemaphoreType.DMA((2, N_CHUNK)),
            pltpu.SemaphoreType.DMA((2, N_CHUNK)),
        ],
        compiler_params=pltpu.CompilerParams(collective_id=0),
    )(table, idx)
